# Initial kernel scaffold; baseline (speedup 1.0000x reference)
import jax
import jax.numpy as jnp
from jax.experimental import pallas as pl


def kernel(points, conv_w, conv_b, bn_gamma, bn_beta, cm_w):
    raise NotImplementedError("write your pallas kernel here")


# speedup vs baseline: 5.6205x; 5.6205x over previous
import functools

import jax
import jax.numpy as jnp
import numpy as np
from jax.experimental import pallas as pl

GS = 8
KNB = 32
RADIUS = 3.0
MARGIN = 4.0
EMBED = 256
EPS = 1e-5

B = 4
N = 16384
M = GS * GS * GS

MB = 128
MBLKS = M // MB
NC = 512
NCH = N // NC

RB = 128
TSTEPS = (B * M) // RB
NTOT = float(B * M * KNB)

_INTERPRET = False

_TRIU = np.triu(np.ones((NC, NC), np.float32))
_AVG = np.kron(np.eye(RB, dtype=np.float32), np.ones((1, KNB), np.float32))


def _minmax_kernel(pts_ref, mn_ref, mx_ref):
    p = pts_ref[...]
    mn_ref[...] = jnp.min(p, axis=1, keepdims=True)
    mx_ref[...] = jnp.max(p, axis=1, keepdims=True)


def _ballq_kernel(ctr_ref, pts_ref, triu_ref, idx_ref, gx_ref, gy_ref,
                  gz_ref):
    cx = ctr_ref[:, 0:1]
    cy = ctr_ref[:, 1:2]
    cz = ctr_ref[:, 2:3]
    triu = triu_ref[...]
    lane = jax.lax.broadcasted_iota(jnp.int32, (1, NC), 1).astype(jnp.float32)

    def body(c, st):
        cnt, ia, ax, ay, az = st
        base = c * NC
        px = pts_ref[0, 0:1, pl.ds(base, NC)]
        py = pts_ref[0, 1:2, pl.ds(base, NC)]
        pz = pts_ref[0, 2:3, pl.ds(base, NC)]
        dx = cx - px
        dy = cy - py
        dz = cz - pz
        d2 = (dx * dx + dy * dy) + dz * dz
        m = (d2 < RADIUS * RADIUS).astype(jnp.float32)
        rank = jax.lax.dot(m, triu,
                           preferred_element_type=jnp.float32) + cnt
        rs = m * jnp.where(rank <= KNB, rank, 0.0)
        ids = lane + jnp.float32(1.0) * base
        icols, xcols, ycols, zcols = [], [], [], []
        for k in range(KNB):
            eq = (rs == np.float32(k + 1)).astype(jnp.float32)
            icols.append(jnp.sum(eq * ids, axis=1, keepdims=True))
            xcols.append(jnp.sum(eq * px, axis=1, keepdims=True))
            ycols.append(jnp.sum(eq * py, axis=1, keepdims=True))
            zcols.append(jnp.sum(eq * pz, axis=1, keepdims=True))
        ia = ia + jnp.concatenate(icols, axis=1)
        ax = ax + jnp.concatenate(xcols, axis=1)
        ay = ay + jnp.concatenate(ycols, axis=1)
        az = az + jnp.concatenate(zcols, axis=1)
        cnt = cnt + jnp.sum(m, axis=1, keepdims=True)
        return cnt, ia, ax, ay, az

    z = jnp.zeros((MB, KNB), jnp.float32)
    cnt0 = jnp.zeros((MB, 1), jnp.float32)
    cnt, ia, ax, ay, az = jax.lax.fori_loop(0, NCH, body,
                                            (cnt0, z, z, z, z))

    slot = jax.lax.broadcasted_iota(jnp.int32, (MB, KNB), 1).astype(jnp.float32)
    valid = slot < cnt
    idx_ref[...] = jnp.where(valid, ia, -1.0).astype(jnp.int32)
    gx_ref[...] = ax
    gy_ref[...] = ay
    gz_ref[...] = az


def _feat8(p_ref, c_ref):
    p3 = p_ref[...]
    c3 = c_ref[...]
    pad = jnp.all(p3 == 0.0, axis=1, keepdims=True)
    rel = jnp.where(pad, 0.0, p3 - c3)
    x8 = jnp.concatenate(
        [rel, p3, jnp.zeros((RB * KNB, 2), jnp.float32)], axis=1)
    return x8.astype(jnp.bfloat16)


def _stats_kernel(p_ref, c_ref, wt_ref, cb_ref, ssum_ref, ssq_ref):
    x8 = _feat8(p_ref, c_ref)
    y = jax.lax.dot(x8, wt_ref[...],
                    preferred_element_type=jnp.float32) + cb_ref[...]
    s1 = jnp.sum(y, axis=0, keepdims=True)
    s2 = jnp.sum(y * y, axis=0, keepdims=True)
    first = pl.program_id(0) == 0
    ssum_ref[...] = jnp.where(first, s1, ssum_ref[...] + s1)
    ssq_ref[...] = jnp.where(first, s2, ssq_ref[...] + s2)


def _apply_kernel(p_ref, c_ref, wt_ref, cb_ref, gam_ref, bet_ref,
                  cm_ref, avg_ref, ssum_ref, ssq_ref, z_ref):
    x8 = _feat8(p_ref, c_ref)
    y = jax.lax.dot(x8, wt_ref[...],
                    preferred_element_type=jnp.float32) + cb_ref[...]
    mean = ssum_ref[...] * np.float32(1.0 / NTOT)
    var = ssq_ref[...] * np.float32(1.0 / NTOT) - mean * mean
    yn = (y - mean) / jnp.sqrt(var + EPS) * gam_ref[...] + bet_ref[...]
    yr = jnp.maximum(yn, 0.0)
    ym = jax.lax.dot(avg_ref[...], yr,
                     preferred_element_type=jnp.float32,
                     precision=jax.lax.Precision.HIGHEST) * np.float32(1.0 / KNB)
    z_ref[...] = jax.lax.dot(ym.astype(jnp.bfloat16), cm_ref[...],
                             preferred_element_type=jnp.float32)


def _minmax(points):
    return pl.pallas_call(
        _minmax_kernel,
        grid=(B,),
        in_specs=[pl.BlockSpec((1, N, 3), lambda b: (b, 0, 0))],
        out_specs=[pl.BlockSpec((1, 1, 3), lambda b: (b, 0, 0)),
                   pl.BlockSpec((1, 1, 3), lambda b: (b, 0, 0))],
        out_shape=[jax.ShapeDtypeStruct((B, 1, 3), jnp.float32),
                   jax.ShapeDtypeStruct((B, 1, 3), jnp.float32)],
        interpret=_INTERPRET,
    )(points)


def _ball_query(centers_flat, points_t):
    return pl.pallas_call(
        _ballq_kernel,
        grid=(B, MBLKS),
        in_specs=[
            pl.BlockSpec((MB, 3), lambda b, mb: (b * MBLKS + mb, 0)),
            pl.BlockSpec((1, 3, N), lambda b, mb: (b, 0, 0)),
            pl.BlockSpec((NC, NC), lambda b, mb: (0, 0)),
        ],
        out_specs=[
            pl.BlockSpec((MB, KNB), lambda b, mb: (b * MBLKS + mb, 0)),
            pl.BlockSpec((MB, KNB), lambda b, mb: (b * MBLKS + mb, 0)),
            pl.BlockSpec((MB, KNB), lambda b, mb: (b * MBLKS + mb, 0)),
            pl.BlockSpec((MB, KNB), lambda b, mb: (b * MBLKS + mb, 0)),
        ],
        out_shape=[
            jax.ShapeDtypeStruct((B * M, KNB), jnp.int32),
            jax.ShapeDtypeStruct((B * M, KNB), jnp.float32),
            jax.ShapeDtypeStruct((B * M, KNB), jnp.float32),
            jax.ShapeDtypeStruct((B * M, KNB), jnp.float32),
        ],
        interpret=_INTERPRET,
    )(centers_flat, points_t, jnp.asarray(_TRIU))


def _offset_net(pflat, cflat, conv_w, conv_b, bn_gamma, bn_beta, cm_w):
    wt = jnp.concatenate(
        [conv_w.T, jnp.zeros((2, EMBED), jnp.float32)],
        axis=0).astype(jnp.bfloat16)
    cb = conv_b.reshape(1, EMBED)
    gam = bn_gamma.reshape(1, EMBED)
    bet = bn_beta.reshape(1, EMBED)
    cm = cm_w.T.astype(jnp.bfloat16)
    row = pl.BlockSpec((RB * KNB, 3), lambda t: (t, 0))
    full = lambda shp: pl.BlockSpec(shp, lambda t: (0, 0))
    ssum, ssq = pl.pallas_call(
        _stats_kernel,
        grid=(TSTEPS,),
        in_specs=[row, row, full((8, EMBED)), full((1, EMBED))],
        out_specs=[full((1, EMBED)), full((1, EMBED))],
        out_shape=[jax.ShapeDtypeStruct((1, EMBED), jnp.float32),
                   jax.ShapeDtypeStruct((1, EMBED), jnp.float32)],
        interpret=_INTERPRET,
    )(pflat, cflat, wt, cb)
    (z,) = pl.pallas_call(
        _apply_kernel,
        grid=(TSTEPS,),
        in_specs=[row, row, full((8, EMBED)), full((1, EMBED)),
                  full((1, EMBED)), full((1, EMBED)), full((EMBED, 3)),
                  full((RB, RB * KNB)), full((1, EMBED)), full((1, EMBED))],
        out_specs=[pl.BlockSpec((RB, 3), lambda t: (t, 0))],
        out_shape=[jax.ShapeDtypeStruct((B * M, 3), jnp.float32)],
        interpret=_INTERPRET,
    )(pflat, cflat, wt, cb, gam, bet, cm, jnp.asarray(_AVG), ssum, ssq)
    return z


def kernel(points, conv_w, conv_b, bn_gamma, bn_beta, cm_w):
    points = points.astype(jnp.float32)
    minc, maxc = _minmax(points)
    lin = jnp.linspace(0.0, 1.0, GS)
    gx_, gy_, gz_ = jnp.meshgrid(lin, lin, lin, indexing='ij')
    grid = jnp.stack([gx_, gy_, gz_], axis=-1).reshape(-1, 3)[None]
    centers = minc + MARGIN + grid * (maxc - minc - 2 * MARGIN)

    points_t = jnp.transpose(points, (0, 2, 1))
    centers_flat = centers.reshape(B * M, 3)

    _, g1x, g1y, g1z = _ball_query(centers_flat, points_t)
    pflat = jnp.stack([g1x, g1y, g1z], axis=-1).reshape(B * M * KNB, 3)
    cflat = jnp.broadcast_to(centers_flat[:, None, :],
                             (B * M, KNB, 3)).reshape(B * M * KNB, 3)
    z = _offset_net(pflat, cflat, conv_w, conv_b, bn_gamma, bn_beta, cm_w)
    offsets = jnp.tanh(z.reshape(B, M, 3)) * MARGIN
    new_centers = centers + offsets
    clamped = jnp.maximum(jnp.minimum(new_centers, maxc), minc)

    idx2, g2x, g2y, g2z = _ball_query(clamped.reshape(B * M, 3), points_t)
    final_cluster = jnp.stack([g2x, g2y, g2z], axis=-1).reshape(B, M, KNB, 3)
    idx = idx2.reshape(B, M, KNB)
    return clamped, final_cluster, idx
